# baseline (speedup 1.0000x reference)
import functools

import jax
import jax.numpy as jnp
from jax import lax
from jax.experimental import pallas as pl
from jax.experimental.pallas import tpu as pltpu
from jax.experimental.pallas import tpu_sc as plsc

N_NODES = 10000
D_IN = 128
D_OUT = 128
N_EDGES = 320000

NC, NS, L = 2, 16, 16
NW = NC * NS
NPAD = 10240
EPW = N_EDGES // NW
CHUNK = 128
EPWP = 10240
NCH = EPWP // CHUNK
TRASH = N_NODES
RPT = NPAD // NS

_MESH = plsc.VectorSubcoreMesh(
    core_axis_name="c", subcore_axis_name="s", num_cores=NC, num_subcores=NS)


NROW = NPAD // 128


@functools.partial(
    pl.kernel,
    out_type=jax.ShapeDtypeStruct((NC, NROW, 128), jnp.float32),
    mesh=_MESH,
    scratch_types=[
        pltpu.VMEM((EPW,), jnp.int32),
        pltpu.VMEM((NROW, 128), jnp.float32),
        pltpu.VMEM((NROW,), jnp.int32),
        pltpu.VMEM_SHARED((NROW, 128), jnp.float32),
    ],
    compiler_params=pltpu.CompilerParams(needs_layout_passes=False),
)
def _deg_kernel(dst_hbm, zeros_hbm, deg_hbm, didx_v, hist_v, rid_v, deg_sp):
    core = lax.axis_index("c")
    sid = lax.axis_index("s")
    wid = sid * NC + core

    one16 = jnp.ones((L,), jnp.float32)

    pltpu.sync_copy(zeros_hbm, hist_v)

    @pl.when(sid == 0)
    def _():
        pltpu.sync_copy(zeros_hbm, deg_sp)

    for i in range(NROW // L):
        rid_v[pl.ds(i * L, L)] = lax.iota(jnp.int32, L) + i * L

    pltpu.sync_copy(dst_hbm.at[wid], didx_v)

    def _hist(i, _):
        idx = didx_v[pl.ds(i * L, L)]
        plsc.addupdate_scatter(
            hist_v,
            [lax.shift_right_logical(idx, 7), lax.bitwise_and(idx, 127)],
            one16)
        return 0
    lax.fori_loop(0, EPW // L, _hist, 0)

    plsc.subcore_barrier()
    pltpu.sync_copy(hist_v, deg_sp.at[rid_v], add=True)
    plsc.subcore_barrier()

    @pl.when(sid == 0)
    def _():
        pltpu.sync_copy(deg_sp, deg_hbm.at[core])


def _scale_body(pt_ref, x_ref, d_ref, xs_ref):
    deg = pt_ref[:, 0:1] + pt_ref[:, 1:2] + 1.0
    dv = lax.rsqrt(deg)
    d_ref[...] = dv
    xs_ref[...] = x_ref[...] * dv


_ROWB = 400


def _scale(pt, x):
    return pl.pallas_call(
        _scale_body,
        grid=(N_NODES // _ROWB,),
        in_specs=[
            pl.BlockSpec((_ROWB, 2), lambda i: (i, 0)),
            pl.BlockSpec((_ROWB, D_IN), lambda i: (i, 0)),
        ],
        out_specs=[
            pl.BlockSpec((_ROWB, 1), lambda i: (i, 0)),
            pl.BlockSpec((_ROWB, D_IN), lambda i: (i, 0)),
        ],
        out_shape=[
            jax.ShapeDtypeStruct((N_NODES, 1), jnp.float32),
            jax.ShapeDtypeStruct((N_NODES, D_IN), jnp.float32),
        ],
    )(pt, x)


@functools.partial(
    pl.kernel,
    out_type=jax.ShapeDtypeStruct((NC, NPAD, D_IN), jnp.float32),
    mesh=_MESH,
    scratch_types=[
        pltpu.VMEM((2, 2, CHUNK), jnp.int32),
        pltpu.VMEM((2, CHUNK, D_IN), jnp.float32),
        pltpu.VMEM_SHARED((NPAD, D_IN), jnp.float32),
        pltpu.SemaphoreType.DMA,
        pltpu.SemaphoreType.DMA,
        pltpu.SemaphoreType.DMA,
        pltpu.SemaphoreType.DMA,
    ],
    compiler_params=pltpu.CompilerParams(needs_layout_passes=False),
)
def _agg_kernel(xs_hbm, idx_hbm, zeros_hbm, t_hbm,
                idx_v, rows_v, t_sp, semi0, semi1, semg0, semg1):
    core = lax.axis_index("c")
    sid = lax.axis_index("s")
    wid = sid * NC + core

    pltpu.sync_copy(zeros_hbm, t_sp.at[pl.ds(sid * RPT, RPT)])
    plsc.subcore_barrier()

    pltpu.async_copy(idx_hbm.at[wid, 0], idx_v.at[0], semi0)
    pltpu.make_async_copy(idx_hbm.at[wid, 0], idx_v.at[0], semi0).wait()
    pltpu.async_copy(idx_hbm.at[wid, 1], idx_v.at[1], semi1)
    pltpu.async_copy(xs_hbm.at[idx_v.at[0, 0]], rows_v.at[0], semg0)

    def _step(j, _):
        c0 = 2 * j
        c1 = c0 + 1
        pltpu.make_async_copy(
            idx_hbm.at[wid, c1], idx_v.at[1], semi1).wait()
        pltpu.async_copy(xs_hbm.at[idx_v.at[1, 0]], rows_v.at[1], semg1)
        pltpu.make_async_copy(
            xs_hbm.at[idx_v.at[0, 0]], rows_v.at[0], semg0).wait()
        pltpu.sync_copy(rows_v.at[0], t_sp.at[idx_v.at[0, 1]], add=True)

        @pl.when(c0 + 2 < NCH)
        def _():
            pltpu.async_copy(idx_hbm.at[wid, c0 + 2], idx_v.at[0], semi0)
            pltpu.make_async_copy(
                idx_hbm.at[wid, c0 + 2], idx_v.at[0], semi0).wait()
            pltpu.async_copy(xs_hbm.at[idx_v.at[0, 0]], rows_v.at[0], semg0)

        pltpu.make_async_copy(
            xs_hbm.at[idx_v.at[1, 0]], rows_v.at[1], semg1).wait()
        pltpu.sync_copy(rows_v.at[1], t_sp.at[idx_v.at[1, 1]], add=True)

        @pl.when(c1 + 2 < NCH)
        def _():
            pltpu.async_copy(idx_hbm.at[wid, c1 + 2], idx_v.at[1], semi1)
        return 0
    lax.fori_loop(0, NCH // 2, _step, 0)

    plsc.subcore_barrier()
    pltpu.sync_copy(t_sp.at[pl.ds(sid * RPT, RPT)],
                    t_hbm.at[core, pl.ds(sid * RPT, RPT)])


def _out_body(t0_ref, t1_ref, xs_ref, d_ref, wmu_ref, wls_ref,
              bmu_ref, bls_ref, mu_ref, ls_ref):
    agg = (t0_ref[0] + t1_ref[0] + xs_ref[...]) * d_ref[...]
    mu_ref[...] = jnp.dot(agg, wmu_ref[...],
                          preferred_element_type=jnp.float32,
                          precision=lax.Precision.HIGHEST) + bmu_ref[...]
    ls_ref[...] = jnp.dot(agg, wls_ref[...],
                          preferred_element_type=jnp.float32,
                          precision=lax.Precision.HIGHEST) + bls_ref[...]


def _outputs(t, xs, d, W_mu, b_mu, W_logstd, b_logstd):
    return pl.pallas_call(
        _out_body,
        grid=(N_NODES // _ROWB,),
        in_specs=[
            pl.BlockSpec((1, _ROWB, D_IN), lambda i: (0, i, 0)),
            pl.BlockSpec((1, _ROWB, D_IN), lambda i: (1, i, 0)),
            pl.BlockSpec((_ROWB, D_IN), lambda i: (i, 0)),
            pl.BlockSpec((_ROWB, 1), lambda i: (i, 0)),
            pl.BlockSpec((D_IN, D_OUT), lambda i: (0, 0)),
            pl.BlockSpec((D_IN, D_OUT), lambda i: (0, 0)),
            pl.BlockSpec((1, D_OUT), lambda i: (0, 0)),
            pl.BlockSpec((1, D_OUT), lambda i: (0, 0)),
        ],
        out_specs=[
            pl.BlockSpec((_ROWB, D_OUT), lambda i: (i, 0)),
            pl.BlockSpec((_ROWB, D_OUT), lambda i: (i, 0)),
        ],
        out_shape=[
            jax.ShapeDtypeStruct((N_NODES, D_OUT), jnp.float32),
            jax.ShapeDtypeStruct((N_NODES, D_OUT), jnp.float32),
        ],
    )(t, t, xs, d, W_mu, W_logstd,
      b_mu.reshape(1, D_OUT), b_logstd.reshape(1, D_OUT))


def kernel(x, edge_index, W_mu, b_mu, W_logstd, b_logstd):
    src = edge_index[0].astype(jnp.int32)
    dst = edge_index[1].astype(jnp.int32)

    zeros_row = jnp.zeros((NROW, 128), jnp.float32)
    partial = _deg_kernel(dst.reshape(NW, EPW), zeros_row)
    pt = jnp.transpose(partial.reshape(NC, NPAD))[:N_NODES]
    d, xs = _scale(pt, x)

    pad = EPWP - EPW
    src_p = jnp.concatenate(
        [src.reshape(NW, EPW), jnp.zeros((NW, pad), jnp.int32)], axis=1)
    dst_p = jnp.concatenate(
        [dst.reshape(NW, EPW), jnp.full((NW, pad), TRASH, jnp.int32)], axis=1)
    idx2 = jnp.stack([src_p.reshape(NW, NCH, CHUNK),
                      dst_p.reshape(NW, NCH, CHUNK)], axis=2)

    zeros_blk = jnp.zeros((RPT, D_IN), jnp.float32)
    t = _agg_kernel(xs, idx2, zeros_blk)
    mu, logstd = _outputs(t, xs, d, W_mu, b_mu, W_logstd, b_logstd)
    return (mu, logstd)


# speedup vs baseline: 1.5060x; 1.5060x over previous
import functools

import jax
import jax.numpy as jnp
from jax import lax
from jax.experimental import pallas as pl
from jax.experimental.pallas import tpu as pltpu
from jax.experimental.pallas import tpu_sc as plsc

N_NODES = 10000
D_IN = 128
D_OUT = 128
N_EDGES = 320000

NC, NS, L = 2, 16, 16
NW = NC * NS
NPAD = 10240
EPW = N_EDGES // NW
CHUNK = 80
EPWP = 10080
NCH = EPWP // CHUNK
TRASH = N_NODES
RPT = NPAD // NS

_MESH = plsc.VectorSubcoreMesh(
    core_axis_name="c", subcore_axis_name="s", num_cores=NC, num_subcores=NS)


NROW = NPAD // 128


@functools.partial(
    pl.kernel,
    out_type=jax.ShapeDtypeStruct((NC, NROW, 128), jnp.float32),
    mesh=_MESH,
    scratch_types=[
        pltpu.VMEM((EPW,), jnp.int32),
        pltpu.VMEM((NROW, 128), jnp.float32),
        pltpu.VMEM((NROW,), jnp.int32),
        pltpu.VMEM_SHARED((NROW, 128), jnp.float32),
    ],
    compiler_params=pltpu.CompilerParams(needs_layout_passes=False),
)
def _deg_kernel(dst_hbm, zeros_hbm, deg_hbm, didx_v, hist_v, rid_v, deg_sp):
    core = lax.axis_index("c")
    sid = lax.axis_index("s")
    wid = sid * NC + core

    one16 = jnp.ones((L,), jnp.float32)

    pltpu.sync_copy(zeros_hbm, hist_v)

    @pl.when(sid == 0)
    def _():
        pltpu.sync_copy(zeros_hbm, deg_sp)

    for i in range(NROW // L):
        rid_v[pl.ds(i * L, L)] = lax.iota(jnp.int32, L) + i * L

    pltpu.sync_copy(dst_hbm.at[wid], didx_v)

    def _hist(i, _):
        idx = didx_v[pl.ds(i * L, L)]
        plsc.addupdate_scatter(
            hist_v,
            [lax.shift_right_logical(idx, 7), lax.bitwise_and(idx, 127)],
            one16)
        return 0
    lax.fori_loop(0, EPW // L, _hist, 0)

    plsc.subcore_barrier()
    pltpu.sync_copy(hist_v, deg_sp.at[rid_v], add=True)
    plsc.subcore_barrier()

    @pl.when(sid == 0)
    def _():
        pltpu.sync_copy(deg_sp, deg_hbm.at[core])


def _scale_body(pt_ref, x_ref, d_ref, xs_ref):
    deg = pt_ref[:, 0:1] + pt_ref[:, 1:2] + 1.0
    dv = lax.rsqrt(deg)
    d_ref[...] = dv
    xs_ref[...] = x_ref[...] * dv


_ROWB = 400


def _scale(pt, x):
    return pl.pallas_call(
        _scale_body,
        grid=(N_NODES // _ROWB,),
        in_specs=[
            pl.BlockSpec((_ROWB, 2), lambda i: (i, 0)),
            pl.BlockSpec((_ROWB, D_IN), lambda i: (i, 0)),
        ],
        out_specs=[
            pl.BlockSpec((_ROWB, 1), lambda i: (i, 0)),
            pl.BlockSpec((_ROWB, D_IN), lambda i: (i, 0)),
        ],
        out_shape=[
            jax.ShapeDtypeStruct((N_NODES, 1), jnp.float32),
            jax.ShapeDtypeStruct((N_NODES, D_IN), jnp.float32),
        ],
    )(pt, x)


@functools.partial(
    pl.kernel,
    out_type=jax.ShapeDtypeStruct((NC, NPAD, D_IN), jnp.float32),
    mesh=_MESH,
    scratch_types=[
        pltpu.VMEM((2, 2, CHUNK), jnp.int32),
        pltpu.VMEM((2, CHUNK, D_IN), jnp.float32),
        pltpu.VMEM_SHARED((NPAD, D_IN), jnp.float32),
        pltpu.SemaphoreType.DMA,
        pltpu.SemaphoreType.DMA,
        pltpu.SemaphoreType.DMA,
        pltpu.SemaphoreType.DMA,
    ],
    compiler_params=pltpu.CompilerParams(needs_layout_passes=False),
)
def _agg_kernel(xs_hbm, idx_hbm, zeros_hbm, t_hbm,
                idx_v, rows_v, t_sp, semi0, semi1, semg0, semg1):
    core = lax.axis_index("c")
    sid = lax.axis_index("s")
    wid = sid * NC + core

    pltpu.sync_copy(zeros_hbm, t_sp.at[pl.ds(sid * RPT, RPT)])
    plsc.subcore_barrier()

    pltpu.async_copy(idx_hbm.at[wid, 0], idx_v.at[0], semi0)
    pltpu.make_async_copy(idx_hbm.at[wid, 0], idx_v.at[0], semi0).wait()
    pltpu.async_copy(idx_hbm.at[wid, 1], idx_v.at[1], semi1)
    pltpu.async_copy(xs_hbm.at[idx_v.at[0, 0]], rows_v.at[0], semg0)

    def _step(j, _):
        c0 = 2 * j
        c1 = c0 + 1
        pltpu.make_async_copy(
            idx_hbm.at[wid, c1], idx_v.at[1], semi1).wait()
        pltpu.async_copy(xs_hbm.at[idx_v.at[1, 0]], rows_v.at[1], semg1)
        pltpu.make_async_copy(
            xs_hbm.at[idx_v.at[0, 0]], rows_v.at[0], semg0).wait()
        pltpu.sync_copy(rows_v.at[0], t_sp.at[idx_v.at[0, 1]], add=True)

        @pl.when(c0 + 2 < NCH)
        def _():
            pltpu.async_copy(idx_hbm.at[wid, c0 + 2], idx_v.at[0], semi0)
            pltpu.make_async_copy(
                idx_hbm.at[wid, c0 + 2], idx_v.at[0], semi0).wait()
            pltpu.async_copy(xs_hbm.at[idx_v.at[0, 0]], rows_v.at[0], semg0)

        pltpu.make_async_copy(
            xs_hbm.at[idx_v.at[1, 0]], rows_v.at[1], semg1).wait()
        pltpu.sync_copy(rows_v.at[1], t_sp.at[idx_v.at[1, 1]], add=True)

        @pl.when(c1 + 2 < NCH)
        def _():
            pltpu.async_copy(idx_hbm.at[wid, c1 + 2], idx_v.at[1], semi1)
        return 0
    lax.fori_loop(0, NCH // 2, _step, 0)

    plsc.subcore_barrier()
    pltpu.sync_copy(t_sp.at[pl.ds(sid * RPT, RPT)],
                    t_hbm.at[core, pl.ds(sid * RPT, RPT)])


def _out_body(t0_ref, t1_ref, xs_ref, d_ref, wmu_ref, wls_ref,
              bmu_ref, bls_ref, mu_ref, ls_ref):
    agg = (t0_ref[0] + t1_ref[0] + xs_ref[...]) * d_ref[...]
    mu_ref[...] = jnp.dot(agg, wmu_ref[...],
                          preferred_element_type=jnp.float32,
                          precision=lax.Precision.HIGHEST) + bmu_ref[...]
    ls_ref[...] = jnp.dot(agg, wls_ref[...],
                          preferred_element_type=jnp.float32,
                          precision=lax.Precision.HIGHEST) + bls_ref[...]


def _outputs(t, xs, d, W_mu, b_mu, W_logstd, b_logstd):
    return pl.pallas_call(
        _out_body,
        grid=(N_NODES // _ROWB,),
        in_specs=[
            pl.BlockSpec((1, _ROWB, D_IN), lambda i: (0, i, 0)),
            pl.BlockSpec((1, _ROWB, D_IN), lambda i: (1, i, 0)),
            pl.BlockSpec((_ROWB, D_IN), lambda i: (i, 0)),
            pl.BlockSpec((_ROWB, 1), lambda i: (i, 0)),
            pl.BlockSpec((D_IN, D_OUT), lambda i: (0, 0)),
            pl.BlockSpec((D_IN, D_OUT), lambda i: (0, 0)),
            pl.BlockSpec((1, D_OUT), lambda i: (0, 0)),
            pl.BlockSpec((1, D_OUT), lambda i: (0, 0)),
        ],
        out_specs=[
            pl.BlockSpec((_ROWB, D_OUT), lambda i: (i, 0)),
            pl.BlockSpec((_ROWB, D_OUT), lambda i: (i, 0)),
        ],
        out_shape=[
            jax.ShapeDtypeStruct((N_NODES, D_OUT), jnp.float32),
            jax.ShapeDtypeStruct((N_NODES, D_OUT), jnp.float32),
        ],
    )(t, t, xs, d, W_mu, W_logstd,
      b_mu.reshape(1, D_OUT), b_logstd.reshape(1, D_OUT))


def kernel(x, edge_index, W_mu, b_mu, W_logstd, b_logstd):
    src = edge_index[0].astype(jnp.int32)
    dst = edge_index[1].astype(jnp.int32)

    zeros_row = jnp.zeros((NROW, 128), jnp.float32)
    partial = _deg_kernel(dst.reshape(NW, EPW), zeros_row)
    pt = jnp.transpose(partial.reshape(NC, NPAD))[:N_NODES]
    d, xs = _scale(pt, x)

    pad = EPWP - EPW
    src_p = jnp.concatenate(
        [src.reshape(NW, EPW), jnp.zeros((NW, pad), jnp.int32)], axis=1)
    dst_p = jnp.concatenate(
        [dst.reshape(NW, EPW), jnp.full((NW, pad), TRASH, jnp.int32)], axis=1)
    idx2 = jnp.stack([src_p.reshape(NW, NCH, CHUNK),
                      dst_p.reshape(NW, NCH, CHUNK)], axis=2)

    zeros_blk = jnp.zeros((RPT, D_IN), jnp.float32)
    t = _agg_kernel(xs, idx2, zeros_blk)
    mu, logstd = _outputs(t, xs, d, W_mu, b_mu, W_logstd, b_logstd)
    return (mu, logstd)
